# baseline (speedup 1.0000x reference)
import jax
import jax.numpy as jnp
from jax.experimental import pallas as pl
from jax.experimental.pallas import tpu as pltpu

LAMBDA_COE = 0.5
EPS = 1e-12

D = 1024
N = 8192
B = 8192
SW = 1024
NSB = N // SW
NSLOT = 4


def _body(o0_ref, o1_ref, lab_ref, h_hbm, out_ref, hbuf, sem):
    def start(idx):
        pltpu.make_async_copy(
            h_hbm.at[:, pl.ds(idx * SW, SW)],
            hbuf.at[idx % NSLOT],
            sem.at[idx % NSLOT],
        ).start()

    def wait(idx):
        pltpu.make_async_copy(
            h_hbm.at[:, pl.ds(0, SW)],
            hbuf.at[idx % NSLOT],
            sem.at[idx % NSLOT],
        ).wait()

    start(0)
    start(1)
    start(2)

    ones_row = jnp.ones((1, D), dtype=jnp.bfloat16)
    s_row = jnp.zeros((1, D), dtype=jnp.float32)
    tr_vec = jnp.zeros((1, SW), dtype=jnp.float32)

    for i in range(NSB):
        if i + 3 < NSB:
            start(i + 3)
        wait(i)
        hb = hbuf[i % NSLOT].astype(jnp.bfloat16)
        hsq = hb * hb
        colnorm2 = jax.lax.dot_general(
            ones_row, hsq, (((1,), (0,)), ((), ())),
            preferred_element_type=jnp.float32)
        inv = jax.lax.rsqrt(jnp.maximum(colnorm2, EPS * EPS))
        tr_vec = tr_vec + colnorm2 * inv * inv
        s_row = s_row + jax.lax.dot_general(
            inv.astype(jnp.bfloat16), hb, (((1,), (1,)), ((), ())),
            preferred_element_type=jnp.float32)

    sum_g = jnp.sum(s_row * s_row)
    pair_sum = (sum_g - jnp.sum(tr_vec)) * 0.5
    reg = pair_sum * LAMBDA_COE / (N * (N - 1) / 2)

    o0 = o0_ref[...]
    o1 = o1_ref[...]
    lab = lab_ref[...]
    m = jnp.maximum(o0, o1)
    lse = m + jnp.log(jnp.exp(o0 - m) + jnp.exp(o1 - m))
    chosen = jnp.where(lab == 1, o1, o0)
    ce = jnp.sum(lse - chosen) / B

    out_ref[...] = jnp.reshape(ce + reg, (1, 1))


def kernel(outputs, labels, H):
    o0 = outputs[:, 0].reshape(64, 128)
    o1 = outputs[:, 1].reshape(64, 128)
    lab = labels.astype(jnp.int32).reshape(64, 128)

    out = pl.pallas_call(
        _body,
        in_specs=[
            pl.BlockSpec((64, 128), lambda: (0, 0)),
            pl.BlockSpec((64, 128), lambda: (0, 0)),
            pl.BlockSpec((64, 128), lambda: (0, 0)),
            pl.BlockSpec(memory_space=pl.ANY),
        ],
        out_specs=pl.BlockSpec((1, 1), lambda: (0, 0)),
        out_shape=jax.ShapeDtypeStruct((1, 1), jnp.float32),
        scratch_shapes=[
            pltpu.VMEM((NSLOT, D, SW), jnp.float32),
            pltpu.SemaphoreType.DMA((NSLOT,)),
        ],
    )(o0, o1, lab, H)
    return out[0, 0]


# speedup vs baseline: 1.0427x; 1.0427x over previous
import jax
import jax.numpy as jnp
from jax.experimental import pallas as pl
from jax.experimental.pallas import tpu as pltpu

LAMBDA_COE = 0.5
EPS = 1e-12

D = 1024
N = 8192
B = 8192
SW = 512
NSB = N // SW
NSLOT = 4


def _body(o0_ref, o1_ref, lab_ref, h_hbm, out_ref, hbuf, sem):
    def start(idx):
        pltpu.make_async_copy(
            h_hbm.at[:, pl.ds(idx * SW, SW)],
            hbuf.at[idx % NSLOT],
            sem.at[idx % NSLOT],
        ).start()

    def wait(idx):
        pltpu.make_async_copy(
            h_hbm.at[:, pl.ds(0, SW)],
            hbuf.at[idx % NSLOT],
            sem.at[idx % NSLOT],
        ).wait()

    start(0)
    start(1)
    start(2)

    ones_row = jnp.ones((1, D), dtype=jnp.bfloat16)
    s_row = jnp.zeros((1, D), dtype=jnp.float32)
    tr_vec = jnp.zeros((1, SW), dtype=jnp.float32)

    for i in range(NSB):
        if i + 3 < NSB:
            start(i + 3)
        wait(i)
        hb = hbuf[i % NSLOT].astype(jnp.bfloat16)
        hsq = hb * hb
        colnorm2 = jax.lax.dot_general(
            ones_row, hsq, (((1,), (0,)), ((), ())),
            preferred_element_type=jnp.float32)
        inv = jax.lax.rsqrt(jnp.maximum(colnorm2, EPS * EPS))
        tr_vec = tr_vec + colnorm2 * inv * inv
        s_row = s_row + jax.lax.dot_general(
            inv.astype(jnp.bfloat16), hb, (((1,), (1,)), ((), ())),
            preferred_element_type=jnp.float32)

    sum_g = jnp.sum(s_row * s_row)
    pair_sum = (sum_g - jnp.sum(tr_vec)) * 0.5
    reg = pair_sum * LAMBDA_COE / (N * (N - 1) / 2)

    o0 = o0_ref[...]
    o1 = o1_ref[...]
    lab = lab_ref[...]
    m = jnp.maximum(o0, o1)
    lse = m + jnp.log(jnp.exp(o0 - m) + jnp.exp(o1 - m))
    chosen = jnp.where(lab == 1, o1, o0)
    ce = jnp.sum(lse - chosen) / B

    out_ref[...] = jnp.reshape(ce + reg, (1, 1))


def kernel(outputs, labels, H):
    o0 = outputs[:, 0].reshape(64, 128)
    o1 = outputs[:, 1].reshape(64, 128)
    lab = labels.astype(jnp.int32).reshape(64, 128)

    out = pl.pallas_call(
        _body,
        in_specs=[
            pl.BlockSpec((64, 128), lambda: (0, 0)),
            pl.BlockSpec((64, 128), lambda: (0, 0)),
            pl.BlockSpec((64, 128), lambda: (0, 0)),
            pl.BlockSpec(memory_space=pl.ANY),
        ],
        out_specs=pl.BlockSpec((1, 1), lambda: (0, 0)),
        out_shape=jax.ShapeDtypeStruct((1, 1), jnp.float32),
        scratch_shapes=[
            pltpu.VMEM((NSLOT, D, SW), jnp.float32),
            pltpu.SemaphoreType.DMA((NSLOT,)),
        ],
    )(o0, o1, lab, H)
    return out[0, 0]
